# baseline (speedup 1.0000x reference)
import functools

import jax
import jax.numpy as jnp
from jax import lax
from jax.experimental import pallas as pl
from jax.experimental.pallas import tpu as pltpu
from jax.experimental.pallas import tpu_sc as plsc

E = 64
B = 4
D_OUT = 1024
D_IN = 1024
D = D_OUT * D_IN

L = 16
NC = 2
NS = 16
NW = NC * NS
WPB = NW // B
PART = D // WPB
GR = 16384
ROWS_PER_E = D // GR
G = PART // GR
UNROLL = 8


def _shuf_max(v, sbuf, iota):
    for sh in (1, 2, 4, 8):
        sbuf[...] = v
        v = jnp.maximum(v, plsc.load_gather(sbuf, [iota ^ sh]))
    return v


def _shuf_min_i32(v, sbuf, iota):
    for sh in (1, 2, 4, 8):
        sbuf[...] = v
        v = jnp.minimum(v, plsc.load_gather(sbuf, [iota ^ sh]))
    return v


def _routing(lbuf, fsc, isc, iota):
    vs = [lbuf[pl.ds(j * L, L)] for j in range(E // L)]

    m = vs[0]
    for v in vs[1:]:
        m = jnp.maximum(m, v)
    m1v = _shuf_max(m, fsc, iota)

    cmin = jnp.full((L,), E, jnp.int32)
    for j, v in enumerate(vs):
        cmin = jnp.minimum(cmin, jnp.where(v == m1v, iota + (j * L), E))
    i1v = _shuf_min_i32(cmin, isc, iota)

    neg_inf = jnp.float32(-jnp.inf)
    vs2 = [jnp.where(iota + (j * L) == i1v, neg_inf, v) for j, v in enumerate(vs)]
    m2 = vs2[0]
    for v in vs2[1:]:
        m2 = jnp.maximum(m2, v)
    m2v = _shuf_max(m2, fsc, iota)

    cmin2 = jnp.full((L,), E, jnp.int32)
    for j, v in enumerate(vs2):
        cmin2 = jnp.minimum(cmin2, jnp.where(v == m2v, iota + (j * L), E))
    i2v = _shuf_min_i32(cmin2, isc, iota)

    ev = jnp.exp(m2v - m1v)
    w1v = 1.0 / (1.0 + ev)
    w2v = ev * w1v
    return i1v, i2v, w1v, w2v


def _sc_body(cl_hbm, k_hbm, out_hbm,
             lbuf, fsc, isc, idx0, idx1, x0, x1, ob0, ob1,
             sg0, sg1, so0, so1):
    wid = lax.axis_index("s") * NC + lax.axis_index("c")
    b = wid // WPB
    part = wid & (WPB - 1)

    pltpu.sync_copy(cl_hbm.at[pl.ds(b * E, E)], lbuf)
    iota = lax.iota(jnp.int32, L)
    i1v, i2v, w1v, w2v = _routing(lbuf, fsc, isc, iota)

    span0 = part * G
    rowv0 = jnp.where(iota == 0, i1v, i2v) * ROWS_PER_E + span0
    base_out = b * D + part * PART

    idx = (idx0, idx1)
    x = (x0, x1)
    ob = (ob0, ob1)
    sg = (sg0, sg1)
    so = (so0, so1)

    def issue_gather(g):
        s = g & 1
        idx[s][...] = rowv0 + g
        return pltpu.async_copy(k_hbm.at[idx[s].at[pl.ds(0, 2)]], x[s], sg[s])

    gathers = {0: issue_gather(0)}
    owrites = {}
    for g in range(G):
        s = g & 1
        gathers.pop(g).wait()
        if g + 1 < G:
            gathers[g + 1] = issue_gather(g + 1)
        if g - 2 in owrites:
            owrites.pop(g - 2).wait()

        xs, obs = x[s], ob[s]

        @plsc.parallel_loop(0, GR // L, unroll=UNROLL)
        def _(c):
            a0 = xs[0, pl.ds(c * L, L)]
            a1 = xs[1, pl.ds(c * L, L)]
            obs[pl.ds(c * L, L)] = w1v * a0 + w2v * a1

        owrites[g] = pltpu.async_copy(
            obs, out_hbm.at[pl.ds(base_out + g * GR, GR)], so[s])

    for g in sorted(owrites):
        owrites.pop(g).wait()


_mesh = plsc.VectorSubcoreMesh(core_axis_name="c", subcore_axis_name="s")

_sc_call = functools.partial(
    pl.kernel,
    mesh=_mesh,
    compiler_params=pltpu.CompilerParams(needs_layout_passes=False),
    out_type=jax.ShapeDtypeStruct((B * D,), jnp.float32),
    scratch_types=[
        pltpu.VMEM((E,), jnp.float32),
        pltpu.VMEM((L,), jnp.float32),
        pltpu.VMEM((L,), jnp.int32),
        pltpu.VMEM((L,), jnp.int32),
        pltpu.VMEM((L,), jnp.int32),
        pltpu.VMEM((2, GR), jnp.float32),
        pltpu.VMEM((2, GR), jnp.float32),
        pltpu.VMEM((GR,), jnp.float32),
        pltpu.VMEM((GR,), jnp.float32),
        pltpu.SemaphoreType.DMA,
        pltpu.SemaphoreType.DMA,
        pltpu.SemaphoreType.DMA,
        pltpu.SemaphoreType.DMA,
    ],
)(_sc_body)


def kernel(config_logits, kernel):
    cl_flat = config_logits.reshape(B * E)
    k_rows = kernel.reshape(E * ROWS_PER_E, GR)
    out = _sc_call(cl_flat, k_rows)
    return out.reshape(B, D_OUT, D_IN)


# speedup vs baseline: 5.2487x; 5.2487x over previous
import functools

import jax
import jax.numpy as jnp
from jax import lax
from jax.experimental import pallas as pl
from jax.experimental.pallas import tpu as pltpu
from jax.experimental.pallas import tpu_sc as plsc

E = 64
B = 4
D_OUT = 1024
D_IN = 1024
D = D_OUT * D_IN

L = 16
NC = 2
NS = 16
NW = NC * NS
WPB = NW // B
PART = D // WPB
R = 1024
ROWS_PER_E = D // R
GROUP = L * R
G = PART // GROUP
UNROLL = 8


def _shuf_max(v, sbuf, iota):
    for sh in (1, 2, 4, 8):
        sbuf[...] = v
        v = jnp.maximum(v, plsc.load_gather(sbuf, [iota ^ sh]))
    return v


def _shuf_min_i32(v, sbuf, iota):
    for sh in (1, 2, 4, 8):
        sbuf[...] = v
        v = jnp.minimum(v, plsc.load_gather(sbuf, [iota ^ sh]))
    return v


def _routing(lbuf, fsc, isc, iota):
    vs = [lbuf[pl.ds(j * L, L)] for j in range(E // L)]

    m = vs[0]
    for v in vs[1:]:
        m = jnp.maximum(m, v)
    m1v = _shuf_max(m, fsc, iota)

    cmin = jnp.full((L,), E, jnp.int32)
    for j, v in enumerate(vs):
        cmin = jnp.minimum(cmin, jnp.where(v == m1v, iota + (j * L), E))
    i1v = _shuf_min_i32(cmin, isc, iota)

    neg_inf = jnp.float32(-jnp.inf)
    vs2 = [jnp.where(iota + (j * L) == i1v, neg_inf, v) for j, v in enumerate(vs)]
    m2 = vs2[0]
    for v in vs2[1:]:
        m2 = jnp.maximum(m2, v)
    m2v = _shuf_max(m2, fsc, iota)

    cmin2 = jnp.full((L,), E, jnp.int32)
    for j, v in enumerate(vs2):
        cmin2 = jnp.minimum(cmin2, jnp.where(v == m2v, iota + (j * L), E))
    i2v = _shuf_min_i32(cmin2, isc, iota)

    ev = jnp.exp(m2v - m1v)
    w1v = 1.0 / (1.0 + ev)
    w2v = ev * w1v
    return i1v, i2v, w1v, w2v


def _sc_body(cl_hbm, k_hbm, out_hbm,
             lbuf, fsc, isc, idx_a, idx_b, xa, xb, obuf, sem_a, sem_b):
    wid = lax.axis_index("s") * NC + lax.axis_index("c")
    b = wid // WPB
    part = wid & (WPB - 1)

    pltpu.sync_copy(cl_hbm.at[pl.ds(b * E, E)], lbuf)
    iota = lax.iota(jnp.int32, L)
    i1v, i2v, w1v, w2v = _routing(lbuf, fsc, isc, iota)

    row_a0 = i1v * ROWS_PER_E + part * (PART // R) + iota
    row_b0 = i2v * ROWS_PER_E + part * (PART // R) + iota
    base_out = b * D + part * PART

    def group_body(g, _):
        idx_a[...] = row_a0 + g * L
        idx_b[...] = row_b0 + g * L
        ca = pltpu.async_copy(k_hbm.at[idx_a], xa, sem_a)
        cb = pltpu.async_copy(k_hbm.at[idx_b], xb, sem_b)
        ca.wait()
        cb.wait()

        @plsc.parallel_loop(0, GROUP // L, unroll=UNROLL)
        def _(c):
            r = c >> 6
            col = (c & (R // L - 1)) * L
            a0 = xa[r, pl.ds(col, L)]
            a1 = xb[r, pl.ds(col, L)]
            obuf[pl.ds(r * R + col, L)] = w1v * a0 + w2v * a1

        pltpu.sync_copy(obuf, out_hbm.at[pl.ds(base_out + g * GROUP, GROUP)])
        return 0

    lax.fori_loop(0, G, group_body, 0)


_mesh = plsc.VectorSubcoreMesh(core_axis_name="c", subcore_axis_name="s")

_sc_call = functools.partial(
    pl.kernel,
    mesh=_mesh,
    compiler_params=pltpu.CompilerParams(needs_layout_passes=False),
    out_type=jax.ShapeDtypeStruct((B * D,), jnp.float32),
    scratch_types=[
        pltpu.VMEM((E,), jnp.float32),
        pltpu.VMEM((L,), jnp.float32),
        pltpu.VMEM((L,), jnp.int32),
        pltpu.VMEM((L,), jnp.int32),
        pltpu.VMEM((L,), jnp.int32),
        pltpu.VMEM((L, R), jnp.float32),
        pltpu.VMEM((L, R), jnp.float32),
        pltpu.VMEM((GROUP,), jnp.float32),
        pltpu.SemaphoreType.DMA,
        pltpu.SemaphoreType.DMA,
    ],
)(_sc_body)


def kernel(config_logits, kernel):
    cl_flat = config_logits.reshape(B * E)
    k_rows = kernel.reshape(E * ROWS_PER_E, R)
    out = _sc_call(cl_flat, k_rows)
    return out.reshape(B, D_OUT, D_IN)


# speedup vs baseline: 6.1208x; 1.1661x over previous
import functools

import jax
import jax.numpy as jnp
from jax import lax
from jax.experimental import pallas as pl
from jax.experimental.pallas import tpu as pltpu
from jax.experimental.pallas import tpu_sc as plsc

E = 64
B = 4
D_OUT = 1024
D_IN = 1024
D = D_OUT * D_IN

L = 16
NC = 2
NS = 16
NW = NC * NS
WPB = NW // B
PART = D // WPB
R = 1024
ROWS_PER_E = D // R
GROUP = L * R
G = PART // GROUP
UNROLL = 8


def _shuf_max(v, sbuf, iota):
    for sh in (1, 2, 4, 8):
        sbuf[...] = v
        v = jnp.maximum(v, plsc.load_gather(sbuf, [iota ^ sh]))
    return v


def _shuf_min_i32(v, sbuf, iota):
    for sh in (1, 2, 4, 8):
        sbuf[...] = v
        v = jnp.minimum(v, plsc.load_gather(sbuf, [iota ^ sh]))
    return v


def _routing(lbuf, fsc, isc, iota):
    vs = [lbuf[pl.ds(j * L, L)] for j in range(E // L)]

    m = vs[0]
    for v in vs[1:]:
        m = jnp.maximum(m, v)
    m1v = _shuf_max(m, fsc, iota)

    cmin = jnp.full((L,), E, jnp.int32)
    for j, v in enumerate(vs):
        cmin = jnp.minimum(cmin, jnp.where(v == m1v, iota + (j * L), E))
    i1v = _shuf_min_i32(cmin, isc, iota)

    neg_inf = jnp.float32(-jnp.inf)
    vs2 = [jnp.where(iota + (j * L) == i1v, neg_inf, v) for j, v in enumerate(vs)]
    m2 = vs2[0]
    for v in vs2[1:]:
        m2 = jnp.maximum(m2, v)
    m2v = _shuf_max(m2, fsc, iota)

    cmin2 = jnp.full((L,), E, jnp.int32)
    for j, v in enumerate(vs2):
        cmin2 = jnp.minimum(cmin2, jnp.where(v == m2v, iota + (j * L), E))
    i2v = _shuf_min_i32(cmin2, isc, iota)

    ev = jnp.exp(m2v - m1v)
    w1v = 1.0 / (1.0 + ev)
    w2v = ev * w1v
    return i1v, i2v, w1v, w2v


def _sc_body(cl_hbm, k_hbm, out_hbm,
             lbuf, fsc, isc, idxa0, idxa1, idxb0, idxb1,
             xa0, xa1, xb0, xb1, ob0, ob1,
             sa0, sa1, sb0, sb1, so0, so1):
    wid = lax.axis_index("s") * NC + lax.axis_index("c")
    b = wid // WPB
    part = wid & (WPB - 1)

    pltpu.sync_copy(cl_hbm.at[pl.ds(b * E, E)], lbuf)
    iota = lax.iota(jnp.int32, L)
    i1v, i2v, w1v, w2v = _routing(lbuf, fsc, isc, iota)

    row_a0 = i1v * ROWS_PER_E + part * (PART // R) + iota
    row_b0 = i2v * ROWS_PER_E + part * (PART // R) + iota
    base_out = b * D + part * PART

    idxa = (idxa0, idxa1)
    idxb = (idxb0, idxb1)
    xa = (xa0, xa1)
    xb = (xb0, xb1)
    ob = (ob0, ob1)
    sa = (sa0, sa1)
    sb = (sb0, sb1)
    so = (so0, so1)

    def issue_gathers(g):
        s = g & 1
        idxa[s][...] = row_a0 + g * L
        idxb[s][...] = row_b0 + g * L
        return (pltpu.async_copy(k_hbm.at[idxa[s]], xa[s], sa[s]),
                pltpu.async_copy(k_hbm.at[idxb[s]], xb[s], sb[s]))

    gathers = {0: issue_gathers(0)}
    owrites = {}
    for g in range(G):
        s = g & 1
        ca, cb = gathers.pop(g)
        ca.wait()
        cb.wait()
        if g + 1 < G:
            gathers[g + 1] = issue_gathers(g + 1)
        if g - 2 in owrites:
            owrites.pop(g - 2).wait()

        xas, xbs, obs = xa[s], xb[s], ob[s]

        @plsc.parallel_loop(0, GROUP // L, unroll=UNROLL)
        def _(c):
            r = c >> 6
            col = (c & (R // L - 1)) * L
            a0 = xas[r, pl.ds(col, L)]
            a1 = xbs[r, pl.ds(col, L)]
            obs[pl.ds(r * R + col, L)] = w1v * a0 + w2v * a1

        owrites[g] = pltpu.async_copy(
            obs, out_hbm.at[pl.ds(base_out + g * GROUP, GROUP)], so[s])

    for g in sorted(owrites):
        owrites.pop(g).wait()


_mesh = plsc.VectorSubcoreMesh(core_axis_name="c", subcore_axis_name="s")

_sc_call = functools.partial(
    pl.kernel,
    mesh=_mesh,
    compiler_params=pltpu.CompilerParams(needs_layout_passes=False),
    out_type=jax.ShapeDtypeStruct((B * D,), jnp.float32),
    scratch_types=[
        pltpu.VMEM((E,), jnp.float32),
        pltpu.VMEM((L,), jnp.float32),
        pltpu.VMEM((L,), jnp.int32),
        pltpu.VMEM((L,), jnp.int32),
        pltpu.VMEM((L,), jnp.int32),
        pltpu.VMEM((L,), jnp.int32),
        pltpu.VMEM((L,), jnp.int32),
        pltpu.VMEM((L, R), jnp.float32),
        pltpu.VMEM((L, R), jnp.float32),
        pltpu.VMEM((L, R), jnp.float32),
        pltpu.VMEM((L, R), jnp.float32),
        pltpu.VMEM((GROUP,), jnp.float32),
        pltpu.VMEM((GROUP,), jnp.float32),
        pltpu.SemaphoreType.DMA,
        pltpu.SemaphoreType.DMA,
        pltpu.SemaphoreType.DMA,
        pltpu.SemaphoreType.DMA,
        pltpu.SemaphoreType.DMA,
        pltpu.SemaphoreType.DMA,
    ],
)(_sc_body)


def kernel(config_logits, kernel):
    cl_flat = config_logits.reshape(B * E)
    k_rows = kernel.reshape(E * ROWS_PER_E, R)
    out = _sc_call(cl_flat, k_rows)
    return out.reshape(B, D_OUT, D_IN)


# speedup vs baseline: 6.2639x; 1.0234x over previous
import functools

import jax
import jax.numpy as jnp
from jax import lax
from jax.experimental import pallas as pl
from jax.experimental.pallas import tpu as pltpu
from jax.experimental.pallas import tpu_sc as plsc

E = 64
B = 4
D_OUT = 1024
D_IN = 1024
D = D_OUT * D_IN

L = 16
NC = 2
NS = 16
NW = NC * NS
WPB = NW // B
PART = D // WPB
R = 1024
ROWS_PER_E = D // R
GROUP = L * R
G = PART // GROUP
UNROLL = 8


def _shuf_max(v, sbuf, iota):
    for sh in (1, 2, 4, 8):
        sbuf[...] = v
        v = jnp.maximum(v, plsc.load_gather(sbuf, [iota ^ sh]))
    return v


def _shuf_min_i32(v, sbuf, iota):
    for sh in (1, 2, 4, 8):
        sbuf[...] = v
        v = jnp.minimum(v, plsc.load_gather(sbuf, [iota ^ sh]))
    return v


def _routing(lbuf, fsc, isc, iota):
    vs = [lbuf[pl.ds(j * L, L)] for j in range(E // L)]

    m = vs[0]
    for v in vs[1:]:
        m = jnp.maximum(m, v)
    m1v = _shuf_max(m, fsc, iota)

    cmin = jnp.full((L,), E, jnp.int32)
    for j, v in enumerate(vs):
        cmin = jnp.minimum(cmin, jnp.where(v == m1v, iota + (j * L), E))
    i1v = _shuf_min_i32(cmin, isc, iota)

    neg_inf = jnp.float32(-jnp.inf)
    vs2 = [jnp.where(iota + (j * L) == i1v, neg_inf, v) for j, v in enumerate(vs)]
    m2 = vs2[0]
    for v in vs2[1:]:
        m2 = jnp.maximum(m2, v)
    m2v = _shuf_max(m2, fsc, iota)

    cmin2 = jnp.full((L,), E, jnp.int32)
    for j, v in enumerate(vs2):
        cmin2 = jnp.minimum(cmin2, jnp.where(v == m2v, iota + (j * L), E))
    i2v = _shuf_min_i32(cmin2, isc, iota)

    ev = jnp.exp(m2v - m1v)
    w1v = 1.0 / (1.0 + ev)
    w2v = ev * w1v
    return i1v, i2v, w1v, w2v


def _sc_body(cl_hbm, k_hbm, out_hbm,
             lbuf, fsc, isc, idxa0, idxa1, idxb0, idxb1,
             xa0, xa1, xb0, xb1, ob0, ob1,
             sa0, sa1, sb0, sb1, so0, so1):
    wid = lax.axis_index("s") * NC + lax.axis_index("c")
    b = wid // WPB
    part = wid & (WPB - 1)

    pltpu.sync_copy(cl_hbm.at[pl.ds(b * E, E)], lbuf)
    iota = lax.iota(jnp.int32, L)
    i1v, i2v, w1v, w2v = _routing(lbuf, fsc, isc, iota)

    row_a0 = i1v * ROWS_PER_E + part * (PART // R) + iota
    row_b0 = i2v * ROWS_PER_E + part * (PART // R) + iota
    base_out = b * D + part * PART

    idxa = (idxa0, idxa1)
    idxb = (idxb0, idxb1)
    xa = (xa0, xa1)
    xb = (xb0, xb1)
    ob = (ob0, ob1)
    sa = (sa0, sa1)
    sb = (sb0, sb1)
    so = (so0, so1)

    def issue_gathers(g, s):
        idxa[s][...] = row_a0 + g * L
        idxb[s][...] = row_b0 + g * L
        pltpu.async_copy(k_hbm.at[idxa[s]], xa[s], sa[s])
        pltpu.async_copy(k_hbm.at[idxb[s]], xb[s], sb[s])

    def wait_gathers(s):
        pltpu.make_async_copy(k_hbm.at[idxa[s]], xa[s], sa[s]).wait()
        pltpu.make_async_copy(k_hbm.at[idxb[s]], xb[s], sb[s]).wait()

    def wait_owrite(s):
        pltpu.make_async_copy(
            ob[s], out_hbm.at[pl.ds(base_out, GROUP)], so[s]).wait()

    def compute_group(g, s):
        xas, xbs, obs = xa[s], xb[s], ob[s]

        @plsc.parallel_loop(0, GROUP // L, unroll=UNROLL)
        def _(c):
            r = c >> 6
            col = (c & (R // L - 1)) * L
            a0 = xas[r, pl.ds(col, L)]
            a1 = xbs[r, pl.ds(col, L)]
            obs[pl.ds(r * R + col, L)] = w1v * a0 + w2v * a1

        pltpu.async_copy(
            obs, out_hbm.at[pl.ds(base_out + g * GROUP, GROUP)], so[s])

    issue_gathers(0, 0)

    def pair_body(i, _):
        g = i * 2
        wait_gathers(0)
        issue_gathers(g + 1, 1)

        @pl.when(i > 0)
        def _():
            wait_owrite(0)

        compute_group(g, 0)
        wait_gathers(1)

        @pl.when(i < G // 2 - 1)
        def _():
            issue_gathers(g + 2, 0)

        @pl.when(i > 0)
        def _():
            wait_owrite(1)

        compute_group(g + 1, 1)
        return 0

    lax.fori_loop(0, G // 2, pair_body, 0)
    wait_owrite(0)
    wait_owrite(1)


_mesh = plsc.VectorSubcoreMesh(core_axis_name="c", subcore_axis_name="s")

_sc_call = functools.partial(
    pl.kernel,
    mesh=_mesh,
    compiler_params=pltpu.CompilerParams(needs_layout_passes=False),
    out_type=jax.ShapeDtypeStruct((B * D,), jnp.float32),
    scratch_types=[
        pltpu.VMEM((E,), jnp.float32),
        pltpu.VMEM((L,), jnp.float32),
        pltpu.VMEM((L,), jnp.int32),
        pltpu.VMEM((L,), jnp.int32),
        pltpu.VMEM((L,), jnp.int32),
        pltpu.VMEM((L,), jnp.int32),
        pltpu.VMEM((L,), jnp.int32),
        pltpu.VMEM((L, R), jnp.float32),
        pltpu.VMEM((L, R), jnp.float32),
        pltpu.VMEM((L, R), jnp.float32),
        pltpu.VMEM((L, R), jnp.float32),
        pltpu.VMEM((GROUP,), jnp.float32),
        pltpu.VMEM((GROUP,), jnp.float32),
        pltpu.SemaphoreType.DMA,
        pltpu.SemaphoreType.DMA,
        pltpu.SemaphoreType.DMA,
        pltpu.SemaphoreType.DMA,
        pltpu.SemaphoreType.DMA,
        pltpu.SemaphoreType.DMA,
    ],
)(_sc_body)


def kernel(config_logits, kernel):
    cl_flat = config_logits.reshape(B * E)
    k_rows = kernel.reshape(E * ROWS_PER_E, R)
    out = _sc_call(cl_flat, k_rows)
    return out.reshape(B, D_OUT, D_IN)


# speedup vs baseline: 9.1067x; 1.4538x over previous
import functools

import jax
import jax.numpy as jnp
from jax import lax
from jax.experimental import pallas as pl
from jax.experimental.pallas import tpu as pltpu
from jax.experimental.pallas import tpu_sc as plsc

E = 64
B = 4
D_OUT = 1024
D_IN = 1024
D = D_OUT * D_IN

L = 16
NC = 2
NS = 16
NW = NC * NS
WPB = NW // B
PART = D // WPB
R = 1024
ROWS_PER_E = D // R
GROUP = L * R
G = PART // GROUP
UNROLL = 8


def _shuf_max(v, sbuf, iota):
    for sh in (1, 2, 4, 8):
        sbuf[...] = v
        v = jnp.maximum(v, plsc.load_gather(sbuf, [iota ^ sh]))
    return v


def _shuf_min_i32(v, sbuf, iota):
    for sh in (1, 2, 4, 8):
        sbuf[...] = v
        v = jnp.minimum(v, plsc.load_gather(sbuf, [iota ^ sh]))
    return v


def _routing(lbuf, fsc, isc, iota):
    vs = [lbuf[pl.ds(j * L, L)] for j in range(E // L)]

    m = vs[0]
    for v in vs[1:]:
        m = jnp.maximum(m, v)
    m1v = _shuf_max(m, fsc, iota)

    cmin = jnp.full((L,), E, jnp.int32)
    for j, v in enumerate(vs):
        cmin = jnp.minimum(cmin, jnp.where(v == m1v, iota + (j * L), E))
    i1v = _shuf_min_i32(cmin, isc, iota)

    neg_inf = jnp.float32(-jnp.inf)
    vs2 = [jnp.where(iota + (j * L) == i1v, neg_inf, v) for j, v in enumerate(vs)]
    m2 = vs2[0]
    for v in vs2[1:]:
        m2 = jnp.maximum(m2, v)
    m2v = _shuf_max(m2, fsc, iota)

    cmin2 = jnp.full((L,), E, jnp.int32)
    for j, v in enumerate(vs2):
        cmin2 = jnp.minimum(cmin2, jnp.where(v == m2v, iota + (j * L), E))
    i2v = _shuf_min_i32(cmin2, isc, iota)

    ev = jnp.exp(m2v - m1v)
    w1v = 1.0 / (1.0 + ev)
    w2v = ev * w1v
    return i1v, i2v, w1v, w2v


def _sc_body(cl_hbm, k_hbm, out_hbm,
             lbuf, fsc, isc, idxa0, idxa1, idxb0, idxb1,
             xa0, xa1, xb0, xb1, ob0, ob1,
             sa0, sa1, sb0, sb1, so0, so1):
    wid = lax.axis_index("s") * NC + lax.axis_index("c")
    b = wid // WPB
    part = wid & (WPB - 1)

    pltpu.sync_copy(cl_hbm.at[b], lbuf)
    iota = lax.iota(jnp.int32, L)
    i1v, i2v, w1v, w2v = _routing(lbuf, fsc, isc, iota)

    row_a0 = i1v * ROWS_PER_E + part * (PART // R) + iota
    row_b0 = i2v * ROWS_PER_E + part * (PART // R) + iota
    base_row = part * (PART // R)

    idxa = (idxa0, idxa1)
    idxb = (idxb0, idxb1)
    xa = (xa0, xa1)
    xb = (xb0, xb1)
    ob = (ob0, ob1)
    sa = (sa0, sa1)
    sb = (sb0, sb1)
    so = (so0, so1)

    def issue_gathers(g, s):
        idxa[s][...] = row_a0 + g * L
        idxb[s][...] = row_b0 + g * L
        pltpu.async_copy(k_hbm.at[idxa[s]], xa[s], sa[s])
        pltpu.async_copy(k_hbm.at[idxb[s]], xb[s], sb[s])

    def wait_gathers(s):
        pltpu.make_async_copy(k_hbm.at[idxa[s]], xa[s], sa[s]).wait()
        pltpu.make_async_copy(k_hbm.at[idxb[s]], xb[s], sb[s]).wait()

    def wait_owrite(s):
        pltpu.make_async_copy(
            ob[s], out_hbm.at[b, pl.ds(base_row, L), :], so[s]).wait()

    def compute_group(g, s):
        xas, xbs, obs = xa[s], xb[s], ob[s]

        @plsc.parallel_loop(0, GROUP // L, unroll=UNROLL)
        def _(c):
            r = c >> 6
            col = (c & (R // L - 1)) * L
            a0 = xas[r, pl.ds(col, L)]
            a1 = xbs[r, pl.ds(col, L)]
            obs[r, pl.ds(col, L)] = w1v * a0 + w2v * a1

        pltpu.async_copy(
            obs, out_hbm.at[b, pl.ds(base_row + g * L, L), :], so[s])

    issue_gathers(0, 0)

    def pair_body(i, _):
        g = i * 2
        wait_gathers(0)
        issue_gathers(g + 1, 1)

        @pl.when(i > 0)
        def _():
            wait_owrite(0)

        compute_group(g, 0)
        wait_gathers(1)

        @pl.when(i < G // 2 - 1)
        def _():
            issue_gathers(g + 2, 0)

        @pl.when(i > 0)
        def _():
            wait_owrite(1)

        compute_group(g + 1, 1)
        return 0

    lax.fori_loop(0, G // 2, pair_body, 0)
    wait_owrite(0)
    wait_owrite(1)


_mesh = plsc.VectorSubcoreMesh(core_axis_name="c", subcore_axis_name="s")

_sc_call = functools.partial(
    pl.kernel,
    mesh=_mesh,
    compiler_params=pltpu.CompilerParams(needs_layout_passes=False),
    out_type=jax.ShapeDtypeStruct((B, D_OUT, D_IN), jnp.float32),
    scratch_types=[
        pltpu.VMEM((E,), jnp.float32),
        pltpu.VMEM((L,), jnp.float32),
        pltpu.VMEM((L,), jnp.int32),
        pltpu.VMEM((L,), jnp.int32),
        pltpu.VMEM((L,), jnp.int32),
        pltpu.VMEM((L,), jnp.int32),
        pltpu.VMEM((L,), jnp.int32),
        pltpu.VMEM((L, R), jnp.float32),
        pltpu.VMEM((L, R), jnp.float32),
        pltpu.VMEM((L, R), jnp.float32),
        pltpu.VMEM((L, R), jnp.float32),
        pltpu.VMEM((L, R), jnp.float32),
        pltpu.VMEM((L, R), jnp.float32),
        pltpu.SemaphoreType.DMA,
        pltpu.SemaphoreType.DMA,
        pltpu.SemaphoreType.DMA,
        pltpu.SemaphoreType.DMA,
        pltpu.SemaphoreType.DMA,
        pltpu.SemaphoreType.DMA,
    ],
)(_sc_body)


def kernel(config_logits, kernel):
    k_rows = kernel.reshape(E * ROWS_PER_E, R)
    return _sc_call(config_logits, k_rows)
